# baseline (speedup 1.0000x reference)
import functools

import jax
import jax.numpy as jnp
from jax.experimental import pallas as pl
from jax.experimental.pallas import tpu as pltpu
from jax.experimental.pallas import tpu_sc as plsc

NUM_TOKENS = 8192
NUM_CODES = 8192
DIM = 256

BM = 512
BN = 2048


def _codes_body(x_ref, et_ref, e_ref, codes_ref, l2e_ref):

    @pl.when(pl.program_id(0) == 0)
    def _():
        l2e_ref[...] = jnp.sum(e_ref[...] * e_ref[...], axis=1)[None, :]

    x = x_ref[...]

    def body(n, carry):
        best_val, best_idx = carry
        et = et_ref[:, pl.ds(n * BN, BN)]
        neg2dot = jax.lax.dot_general(
            x, et, (((1,), (0,)), ((), ())),
            preferred_element_type=jnp.float32)
        l2e = l2e_ref[0, pl.ds(n * BN, BN)]
        scores = l2e[None, :] + neg2dot
        blk_min = jnp.min(scores, axis=1)
        iota = jax.lax.broadcasted_iota(jnp.int32, (BM, BN), 1)
        blk_arg = jnp.min(
            jnp.where(scores <= blk_min[:, None], iota, NUM_CODES),
            axis=1) + n * BN
        take = blk_min < best_val
        return (jnp.where(take, blk_min, best_val),
                jnp.where(take, blk_arg, best_idx))

    init = (jnp.full((BM,), jnp.inf, jnp.float32),
            jnp.zeros((BM,), jnp.int32))
    _, best_idx = jax.lax.fori_loop(0, NUM_CODES // BN, body, init)
    codes_ref[...] = best_idx


def _compute_codes(x_bf, et_bf, embeddings, interpret=False):
    return pl.pallas_call(
        _codes_body,
        grid=(NUM_TOKENS // BM,),
        in_specs=[
            pl.BlockSpec((BM, DIM), lambda i: (i, 0)),
            pl.BlockSpec((DIM, NUM_CODES), lambda i: (0, 0)),
            pl.BlockSpec((NUM_CODES, DIM), lambda i: (0, 0)),
        ],
        out_specs=pl.BlockSpec((BM,), lambda i: (i,)),
        out_shape=jax.ShapeDtypeStruct((NUM_TOKENS,), jnp.int32),
        scratch_shapes=[pltpu.VMEM((1, NUM_CODES), jnp.float32)],
        interpret=interpret,
    )(x_bf, et_bf, embeddings)


def _make_sc_gather():
    info = plsc.get_sparse_core_info()
    nw = info.num_cores * info.num_subcores
    b_per_w = NUM_TOKENS // nw
    mesh = plsc.VectorSubcoreMesh(core_axis_name="c", subcore_axis_name="s")

    @functools.partial(
        pl.kernel, mesh=mesh,
        out_type=jax.ShapeDtypeStruct((NUM_TOKENS, DIM), jnp.float32),
        scratch_types=[
            pltpu.VMEM((b_per_w,), jnp.int32),
            pltpu.VMEM((b_per_w, DIM), jnp.float32),
            pltpu.SemaphoreType.DMA,
        ],
    )
    def gather(table_hbm, idx_hbm, out_hbm, idx_v, rows_v, sem):
        wid = jax.lax.axis_index("s") * info.num_cores + jax.lax.axis_index("c")
        base = wid * b_per_w
        pltpu.sync_copy(idx_hbm.at[pl.ds(base, b_per_w)], idx_v)
        pltpu.async_copy(table_hbm.at[idx_v], rows_v, sem).wait()
        pltpu.sync_copy(rows_v, out_hbm.at[pl.ds(base, b_per_w)])

    return gather


def kernel(inputs, embeddings):
    b, m, _, d = inputs.shape
    x = inputs.reshape(-1, d)
    x_bf = (-2.0 * x).astype(jnp.bfloat16)
    et_bf = embeddings.T.astype(jnp.bfloat16)
    codes_flat = _compute_codes(x_bf, et_bf, embeddings)
    code_vecs_flat = _make_sc_gather()(embeddings, codes_flat)
    return (codes_flat.reshape(b, m, m),
            code_vecs_flat.reshape(b, m, m, d))


# speedup vs baseline: 1.0724x; 1.0724x over previous
import functools

import jax
import jax.numpy as jnp
from jax.experimental import pallas as pl
from jax.experimental.pallas import tpu as pltpu
from jax.experimental.pallas import tpu_sc as plsc

NUM_TOKENS = 8192
NUM_CODES = 8192
DIM = 256

BM = 512
BN = 2048


def _codes_body(x_ref, e_ref, codes_ref, l2e_ref, et_ref):

    l2e_ref[...] = jnp.sum(e_ref[...] * e_ref[...], axis=1)[None, :]
    et_ref[...] = e_ref[...].T.astype(jnp.bfloat16)

    def m_body(m, _):
        x = (-2.0 * x_ref[pl.ds(m * BM, BM), :]).astype(jnp.bfloat16)

        def body(n, carry):
            best_val, best_idx = carry
            et = et_ref[:, pl.ds(n * BN, BN)]
            neg2dot = jax.lax.dot_general(
                x, et, (((1,), (0,)), ((), ())),
                preferred_element_type=jnp.float32)
            l2e = l2e_ref[0, pl.ds(n * BN, BN)]
            scores = l2e[None, :] + neg2dot
            blk_min = jnp.min(scores, axis=1)
            iota = jax.lax.broadcasted_iota(jnp.int32, (BM, BN), 1)
            blk_arg = jnp.min(
                jnp.where(scores <= blk_min[:, None], iota, NUM_CODES),
                axis=1) + n * BN
            take = blk_min < best_val
            return (jnp.where(take, blk_min, best_val),
                    jnp.where(take, blk_arg, best_idx))

        init = (jnp.full((BM,), jnp.inf, jnp.float32),
                jnp.zeros((BM,), jnp.int32))
        _, best_idx = jax.lax.fori_loop(0, NUM_CODES // BN, body, init)
        codes_ref[pl.ds(m * BM, BM)] = best_idx
        return 0

    jax.lax.fori_loop(0, NUM_TOKENS // BM, m_body, 0)


def _compute_codes(x, embeddings, interpret=False):
    return pl.pallas_call(
        _codes_body,
        out_shape=jax.ShapeDtypeStruct((NUM_TOKENS,), jnp.int32),
        scratch_shapes=[
            pltpu.VMEM((1, NUM_CODES), jnp.float32),
            pltpu.VMEM((DIM, NUM_CODES), jnp.bfloat16),
        ],
        interpret=interpret,
    )(x, embeddings)


def _make_sc_gather():
    info = plsc.get_sparse_core_info()
    nw = info.num_cores * info.num_subcores
    b_per_w = NUM_TOKENS // nw
    mesh = plsc.VectorSubcoreMesh(core_axis_name="c", subcore_axis_name="s")

    @functools.partial(
        pl.kernel, mesh=mesh,
        out_type=jax.ShapeDtypeStruct((NUM_TOKENS, DIM), jnp.float32),
        scratch_types=[
            pltpu.VMEM((b_per_w,), jnp.int32),
            pltpu.VMEM((b_per_w, DIM), jnp.float32),
            pltpu.SemaphoreType.DMA,
        ],
    )
    def gather(table_hbm, idx_hbm, out_hbm, idx_v, rows_v, sem):
        wid = jax.lax.axis_index("s") * info.num_cores + jax.lax.axis_index("c")
        base = wid * b_per_w
        pltpu.sync_copy(idx_hbm.at[pl.ds(base, b_per_w)], idx_v)
        pltpu.async_copy(table_hbm.at[idx_v], rows_v, sem).wait()
        pltpu.sync_copy(rows_v, out_hbm.at[pl.ds(base, b_per_w)])

    return gather


def kernel(inputs, embeddings):
    b, m, _, d = inputs.shape
    x = inputs.reshape(-1, d)
    codes_flat = _compute_codes(x, embeddings)
    code_vecs_flat = _make_sc_gather()(embeddings, codes_flat)
    return (codes_flat.reshape(b, m, m),
            code_vecs_flat.reshape(b, m, m, d))


# speedup vs baseline: 1.3271x; 1.2375x over previous
import functools

import jax
import jax.numpy as jnp
from jax.experimental import pallas as pl
from jax.experimental.pallas import tpu as pltpu
from jax.experimental.pallas import tpu_sc as plsc

NUM_TOKENS = 8192
NUM_CODES = 8192
DIM = 256

BM = 512
BN = 2048


def _codes_body(x_ref, e_ref, codes_ref, l2e_ref, et_ref):

    l2e_ref[...] = jnp.sum(e_ref[...] * e_ref[...], axis=1)[None, :]
    et_ref[...] = e_ref[...].T.astype(jnp.bfloat16)

    def m_body(m, _):
        x = (-2.0 * x_ref[pl.ds(m * BM, BM), :]).astype(jnp.bfloat16)

        best_val = jnp.full((BM,), jnp.inf, jnp.float32)
        best_idx = jnp.zeros((BM,), jnp.int32)
        for n in range(NUM_CODES // BN):
            et = et_ref[:, n * BN:(n + 1) * BN]
            neg2dot = jax.lax.dot_general(
                x, et, (((1,), (0,)), ((), ())),
                preferred_element_type=jnp.float32)
            l2e = l2e_ref[0, n * BN:(n + 1) * BN]
            scores = l2e[None, :] + neg2dot
            blk_min = jnp.min(scores, axis=1)
            blk_arg = jnp.argmin(scores, axis=1).astype(jnp.int32) + n * BN
            take = blk_min < best_val
            best_val = jnp.where(take, blk_min, best_val)
            best_idx = jnp.where(take, blk_arg, best_idx)
        codes_ref[pl.ds(m * BM, BM)] = best_idx
        return 0

    jax.lax.fori_loop(0, NUM_TOKENS // BM, m_body, 0)


def _compute_codes(x, embeddings, interpret=False):
    return pl.pallas_call(
        _codes_body,
        out_shape=jax.ShapeDtypeStruct((NUM_TOKENS,), jnp.int32),
        scratch_shapes=[
            pltpu.VMEM((1, NUM_CODES), jnp.float32),
            pltpu.VMEM((DIM, NUM_CODES), jnp.bfloat16),
        ],
        interpret=interpret,
    )(x, embeddings)


def _make_sc_gather():
    info = plsc.get_sparse_core_info()
    nw = info.num_cores * info.num_subcores
    b_per_w = NUM_TOKENS // nw
    mesh = plsc.VectorSubcoreMesh(core_axis_name="c", subcore_axis_name="s")

    @functools.partial(
        pl.kernel, mesh=mesh,
        out_type=jax.ShapeDtypeStruct((NUM_TOKENS, DIM), jnp.float32),
        scratch_types=[
            pltpu.VMEM((b_per_w,), jnp.int32),
            pltpu.VMEM((b_per_w, DIM), jnp.float32),
            pltpu.SemaphoreType.DMA,
        ],
    )
    def gather(table_hbm, idx_hbm, out_hbm, idx_v, rows_v, sem):
        wid = jax.lax.axis_index("s") * info.num_cores + jax.lax.axis_index("c")
        base = wid * b_per_w
        pltpu.sync_copy(idx_hbm.at[pl.ds(base, b_per_w)], idx_v)
        pltpu.async_copy(table_hbm.at[idx_v], rows_v, sem).wait()
        pltpu.sync_copy(rows_v, out_hbm.at[pl.ds(base, b_per_w)])

    return gather


def kernel(inputs, embeddings):
    b, m, _, d = inputs.shape
    x = inputs.reshape(-1, d)
    codes_flat = _compute_codes(x, embeddings)
    code_vecs_flat = _make_sc_gather()(embeddings, codes_flat)
    return (codes_flat.reshape(b, m, m),
            code_vecs_flat.reshape(b, m, m, d))


# speedup vs baseline: 1.8179x; 1.3698x over previous
import functools

import jax
import jax.numpy as jnp
from jax.experimental import pallas as pl
from jax.experimental.pallas import tpu as pltpu
from jax.experimental.pallas import tpu_sc as plsc

NUM_TOKENS = 8192
NUM_CODES = 8192
DIM = 256

BM = 512
BN = 2048


def _codes_body(x_ref, e_ref, codes_ref, l2e_ref, et_ref):

    l2e_ref[...] = jnp.sum(e_ref[...] * e_ref[...], axis=1)[None, :]
    et_ref[...] = e_ref[...].T.astype(jnp.bfloat16)

    def m_body(m, _):
        x = (-2.0 * x_ref[pl.ds(m * BM, BM), :]).astype(jnp.bfloat16)

        blocks = []
        for n in range(NUM_CODES // BN):
            et = et_ref[:, n * BN:(n + 1) * BN]
            neg2dot = jax.lax.dot_general(
                x, et, (((1,), (0,)), ((), ())),
                preferred_element_type=jnp.float32)
            l2e = l2e_ref[0, n * BN:(n + 1) * BN]
            blocks.append(l2e[None, :] + neg2dot)
        scores = jnp.concatenate(blocks, axis=1)
        best_idx = jnp.argmin(scores, axis=1).astype(jnp.int32)
        codes_ref[pl.ds(m * BM, BM)] = best_idx
        return 0

    jax.lax.fori_loop(0, NUM_TOKENS // BM, m_body, 0)


def _compute_codes(x, embeddings, interpret=False):
    return pl.pallas_call(
        _codes_body,
        out_shape=jax.ShapeDtypeStruct((NUM_TOKENS,), jnp.int32),
        scratch_shapes=[
            pltpu.VMEM((1, NUM_CODES), jnp.float32),
            pltpu.VMEM((DIM, NUM_CODES), jnp.bfloat16),
        ],
        interpret=interpret,
    )(x, embeddings)


def _make_sc_gather():
    info = plsc.get_sparse_core_info()
    nw = info.num_cores * info.num_subcores
    b_per_w = NUM_TOKENS // nw
    mesh = plsc.VectorSubcoreMesh(core_axis_name="c", subcore_axis_name="s")

    @functools.partial(
        pl.kernel, mesh=mesh,
        out_type=jax.ShapeDtypeStruct((NUM_TOKENS, DIM), jnp.float32),
        scratch_types=[
            pltpu.VMEM((b_per_w,), jnp.int32),
            pltpu.VMEM((b_per_w, DIM), jnp.float32),
            pltpu.SemaphoreType.DMA,
        ],
    )
    def gather(table_hbm, idx_hbm, out_hbm, idx_v, rows_v, sem):
        wid = jax.lax.axis_index("s") * info.num_cores + jax.lax.axis_index("c")
        base = wid * b_per_w
        pltpu.sync_copy(idx_hbm.at[pl.ds(base, b_per_w)], idx_v)
        pltpu.async_copy(table_hbm.at[idx_v], rows_v, sem).wait()
        pltpu.sync_copy(rows_v, out_hbm.at[pl.ds(base, b_per_w)])

    return gather


def kernel(inputs, embeddings):
    b, m, _, d = inputs.shape
    x = inputs.reshape(-1, d)
    codes_flat = _compute_codes(x, embeddings)
    code_vecs_flat = _make_sc_gather()(embeddings, codes_flat)
    return (codes_flat.reshape(b, m, m),
            code_vecs_flat.reshape(b, m, m, d))
